# baseline (speedup 1.0000x reference)
import jax
import jax.numpy as jnp
from jax import lax
from jax.experimental import pallas as pl
from jax.experimental.pallas import tpu as pltpu

_P = 4096


def _gather2_kernel(idx_ref, table_ref, out_ref):
    num_hi, wide = table_ref.shape
    d = wide // 4

    table = table_ref[...]
    iota = lax.broadcasted_iota(jnp.int32, (1, num_hi), 1)
    for e in range(2):
        tgt = idx_ref[:, e : e + 1]
        hi = tgt >> 2
        lo = tgt & 3
        onehot = jnp.where(iota == hi, 1.0, 0.0)
        partial = jnp.dot(
            onehot, table, preferred_element_type=jnp.float32
        )
        a = partial[:, : 2 * d]
        b = partial[:, 2 * d :]
        sel1 = jnp.where(lo >= 2, b, a)
        rolled = pltpu.roll(sel1, d, axis=1)
        sel2 = jnp.where((lo & 1) == 0, sel1, rolled)
        out_ref[e : 2 * _P : 2, :] = sel2[:, :d]


def _gather2_call(idx2, table4, num_blocks):
    n_packed = idx2.shape[0]
    num_hi, wide = table4.shape

    return pl.pallas_call(
        _gather2_kernel,
        out_shape=jax.ShapeDtypeStruct((2 * n_packed, wide // 4), jnp.float32),
        grid=(num_blocks,),
        in_specs=[
            pl.BlockSpec((_P, 2), lambda i: (i, 0)),
            pl.BlockSpec((num_hi, wide), lambda i: (0, 0)),
        ],
        out_specs=pl.BlockSpec((2 * _P, wide // 4), lambda i: (i, 0)),
        compiler_params=pltpu.CompilerParams(
            dimension_semantics=("arbitrary",),
            vmem_limit_bytes=64 * 1024 * 1024,
        ),
    )(idx2, table4)


def kernel(user_fea, embedding_location):
    n = user_fea.shape[0]
    num_location, d = embedding_location.shape
    assert num_location % 4 == 0 and d % 2 == 0

    nf = user_fea.shape[1]
    col0 = (jnp.arange(nf, dtype=jnp.int32) == 0).astype(jnp.int32)
    idx2 = jnp.sum(
        user_fea.astype(jnp.int32).reshape(n // 2, 2, nf) * col0, axis=-1
    )
    idx2 = jnp.clip(idx2, 0, num_location - 1)

    rows_per_block = 2 * _P
    n_pad = ((n + rows_per_block - 1) // rows_per_block) * rows_per_block
    if n_pad != n:
        idx2 = jnp.pad(idx2, ((0, (n_pad - n) // 2), (0, 0)))

    table4 = embedding_location.reshape(num_location // 4, 4 * d)

    out = _gather2_call(idx2, table4, (n_pad // 2) // _P)
    return out[:n]


# speedup vs baseline: 1.0025x; 1.0025x over previous
import jax
import jax.numpy as jnp
from jax import lax
from jax.experimental import pallas as pl
from jax.experimental.pallas import tpu as pltpu

_P = 8192


def _gather2_kernel(idx_ref, table_ref, out_ref):
    num_hi, wide = table_ref.shape
    d = wide // 4

    table = table_ref[...]
    iota = lax.broadcasted_iota(jnp.int32, (1, num_hi), 1)
    for e in range(2):
        tgt = idx_ref[:, e : e + 1]
        hi = tgt >> 2
        lo = tgt & 3
        onehot = jnp.where(iota == hi, 1.0, 0.0)
        partial = jnp.dot(
            onehot, table, preferred_element_type=jnp.float32
        )
        a = partial[:, : 2 * d]
        b = partial[:, 2 * d :]
        sel1 = jnp.where(lo >= 2, b, a)
        rolled = pltpu.roll(sel1, d, axis=1)
        sel2 = jnp.where((lo & 1) == 0, sel1, rolled)
        out_ref[e : 2 * _P : 2, :] = sel2[:, :d]


def _gather2_call(idx2, table4, num_blocks):
    n_packed = idx2.shape[0]
    num_hi, wide = table4.shape

    return pl.pallas_call(
        _gather2_kernel,
        out_shape=jax.ShapeDtypeStruct((2 * n_packed, wide // 4), jnp.float32),
        grid=(num_blocks,),
        in_specs=[
            pl.BlockSpec((_P, 2), lambda i: (i, 0)),
            pl.BlockSpec((num_hi, wide), lambda i: (0, 0)),
        ],
        out_specs=pl.BlockSpec((2 * _P, wide // 4), lambda i: (i, 0)),
        compiler_params=pltpu.CompilerParams(
            dimension_semantics=("arbitrary",),
            vmem_limit_bytes=64 * 1024 * 1024,
        ),
    )(idx2, table4)


def kernel(user_fea, embedding_location):
    n = user_fea.shape[0]
    num_location, d = embedding_location.shape
    assert num_location % 4 == 0 and d % 2 == 0

    nf = user_fea.shape[1]
    col0 = (jnp.arange(nf, dtype=jnp.int32) == 0).astype(jnp.int32)
    idx2 = jnp.sum(
        user_fea.astype(jnp.int32).reshape(n // 2, 2, nf) * col0, axis=-1
    )
    idx2 = jnp.clip(idx2, 0, num_location - 1)

    rows_per_block = 2 * _P
    n_pad = ((n + rows_per_block - 1) // rows_per_block) * rows_per_block
    if n_pad != n:
        idx2 = jnp.pad(idx2, ((0, (n_pad - n) // 2), (0, 0)))

    table4 = embedding_location.reshape(num_location // 4, 4 * d)

    out = _gather2_call(idx2, table4, (n_pad // 2) // _P)
    return out[:n]
